# baseline (speedup 1.0000x reference)
import functools

import jax
import jax.numpy as jnp
from jax import lax
from jax.experimental import pallas as pl
from jax.experimental.pallas import tpu as pltpu
from jax.experimental.pallas import tpu_sc as plsc

_D = 1024
_N = 4 * 8192
_NC = 2
_NS = 16
_NW = _NC * _NS
_TPW = _N // _NW
_T = 32
_NCH = _TPW // _T
_NBUF = 3
_PROBE_NO_COMPUTE = False


def _sc_body(x_hbm, ids_hbm, tab_hbm, out_hbm,
             tab_v, ids_v, b0, b1, b2,
             si0, si1, si2, so0, so1, so2):
    bufs = (b0, b1, b2)
    in_sems = (si0, si1, si2)
    out_sems = (so0, so1, so2)
    wid = lax.axis_index("s") * _NC + lax.axis_index("c")
    base = wid * _TPW

    def start_in(c, k):
        pltpu.async_copy(x_hbm.at[pl.ds(base + c * _T, _T)], bufs[k],
                         in_sems[k])

    def wait_in(k):
        pltpu.make_async_copy(x_hbm.at[pl.ds(0, _T)], bufs[k],
                              in_sems[k]).wait()

    def start_out(c, k):
        pltpu.async_copy(bufs[k], out_hbm.at[pl.ds(base + c * _T, _T)],
                         out_sems[k])

    def wait_out(k):
        pltpu.make_async_copy(bufs[k], out_hbm.at[pl.ds(0, _T)],
                              out_sems[k]).wait()

    def compute(c, k):
        buf = bufs[k]
        dnums = lax.GatherDimensionNumbers(
            offset_dims=(), collapsed_slice_dims=(0,), start_index_map=(0,))
        for h in range(_T // 16):
            m16 = ids_v[pl.ds(c * _T + h * 16, 16)].astype(jnp.float32)
            msps = [lax.gather(m16, jnp.full((16, 1), l, jnp.int32), dnums,
                               (1,),
                               mode=lax.GatherScatterMode.PROMISE_IN_BOUNDS)
                    for l in range(16)]

            def jbody(j, carry):
                sl = pl.ds(j * 16, 16)
                r0 = tab_v[sl]
                r1 = tab_v[pl.ds(_D + j * 16, 16)]
                d = r1 - r0
                for l in range(16):
                    buf[h * 16 + l, sl] = (buf[h * 16 + l, sl]
                                           + (r0 + msps[l] * d))
                return carry

            lax.fori_loop(0, _D // 16, jbody, 0)

    def do_chunk(c, k, start_next_in, wait_prev_out=True):
        wait_in(k)
        if not _PROBE_NO_COMPUTE:
            compute(c, k)
        start_out(c, k)
        if start_next_in:
            k2 = (k + 2) % _NBUF
            if wait_prev_out:
                wait_out(k2)
            start_in(c + 2, k2)

    pltpu.sync_copy(tab_hbm, tab_v)
    pltpu.sync_copy(ids_hbm.at[pl.ds(base, _TPW)], ids_v)

    start_in(0, 0)
    start_in(1, 1)
    do_chunk(0, 0, True, wait_prev_out=False)
    do_chunk(1, 1, True)
    do_chunk(2, 2, True)

    def mid(i, carry):
        n = i * _NBUF
        for k in range(_NBUF):
            do_chunk(n + k, k, True)
        return carry

    lax.fori_loop(1, _NCH // _NBUF, mid, 0)

    do_chunk(_NCH - 2, (_NCH - 2) % _NBUF, False)
    do_chunk(_NCH - 1, (_NCH - 1) % _NBUF, False)

    wait_out((_NCH - 3) % _NBUF)
    wait_out((_NCH - 2) % _NBUF)
    wait_out((_NCH - 1) % _NBUF)


@jax.jit
def _sc_call(x2, ids, tab):
    mesh = plsc.VectorSubcoreMesh(core_axis_name="c", subcore_axis_name="s")
    fn = functools.partial(
        pl.kernel,
        mesh=mesh,
        out_type=jax.ShapeDtypeStruct((_N, _D), jnp.float32),
        scratch_types=[
            pltpu.VMEM((2 * _D,), jnp.float32),
            pltpu.VMEM((_TPW,), jnp.int32),
            pltpu.VMEM((_T, _D), jnp.float32),
            pltpu.VMEM((_T, _D), jnp.float32),
            pltpu.VMEM((_T, _D), jnp.float32),
            pltpu.SemaphoreType.DMA,
            pltpu.SemaphoreType.DMA,
            pltpu.SemaphoreType.DMA,
            pltpu.SemaphoreType.DMA,
            pltpu.SemaphoreType.DMA,
            pltpu.SemaphoreType.DMA,
        ],
    )(_sc_body)
    return fn(x2, ids, tab)


def kernel(x, segment_input_ids, seg_table):
    B, S, D = x.shape
    x2 = x.reshape(B * S, D)
    ids = segment_input_ids.astype(jnp.int32).reshape(-1)
    tab = seg_table.reshape(-1)
    out = _sc_call(x2, ids, tab)
    return out.reshape(B, S, D)


# speedup vs baseline: 1.0322x; 1.0322x over previous
import functools

import jax
import jax.numpy as jnp
from jax import lax
from jax.experimental import pallas as pl
from jax.experimental.pallas import tpu as pltpu
from jax.experimental.pallas import tpu_sc as plsc

_D = 1024
_N = 4 * 8192
_NC = 2
_NS = 16
_NW = _NC * _NS
_TPW = _N // _NW
_T = 32
_NCH = _TPW // _T
_NBUF = 3
_PROBE_NO_COMPUTE = False


def _sc_body(x_hbm, ids_hbm, tab_hbm, out_hbm,
             tab_v, ids_v, b0, b1, b2,
             si0, si1, si2, so0, so1, so2):
    bufs = (b0, b1, b2)
    in_sems = (si0, si1, si2)
    out_sems = (so0, so1, so2)
    wid = lax.axis_index("s") * _NC + lax.axis_index("c")
    base = wid * _TPW

    def start_in(c, k):
        pltpu.async_copy(x_hbm.at[pl.ds(base + c * _T, _T)], bufs[k],
                         in_sems[k])

    def wait_in(k):
        pltpu.make_async_copy(x_hbm.at[pl.ds(0, _T)], bufs[k],
                              in_sems[k]).wait()

    def start_out(c, k):
        pltpu.async_copy(bufs[k], out_hbm.at[pl.ds(base + c * _T, _T)],
                         out_sems[k])

    def wait_out(k):
        pltpu.make_async_copy(bufs[k], out_hbm.at[pl.ds(0, _T)],
                              out_sems[k]).wait()

    def compute(c, k):
        buf = bufs[k]
        dnums = lax.GatherDimensionNumbers(
            offset_dims=(), collapsed_slice_dims=(0,), start_index_map=(0,))
        for h in range(_T // 16):
            m16 = ids_v[pl.ds(c * _T + h * 16, 16)].astype(jnp.float32)
            msps = [lax.gather(m16, jnp.full((16, 1), l, jnp.int32), dnums,
                               (1,),
                               mode=lax.GatherScatterMode.PROMISE_IN_BOUNDS)
                    for l in range(16)]

            def jbody(j, carry):
                sl = pl.ds(j * 16, 16)
                r0 = tab_v[sl]
                r1 = tab_v[pl.ds(_D + j * 16, 16)]
                d = r1 - r0
                for l in range(16):
                    plsc.addupdate(buf.at[h * 16 + l, sl], r0 + msps[l] * d)
                return carry

            lax.fori_loop(0, _D // 16, jbody, 0)

    def do_chunk(c, k, start_next_in, wait_prev_out=True):
        wait_in(k)
        if not _PROBE_NO_COMPUTE:
            compute(c, k)
        start_out(c, k)
        if start_next_in:
            k2 = (k + 2) % _NBUF
            if wait_prev_out:
                wait_out(k2)
            start_in(c + 2, k2)

    pltpu.sync_copy(tab_hbm, tab_v)
    pltpu.sync_copy(ids_hbm.at[pl.ds(base, _TPW)], ids_v)

    start_in(0, 0)
    start_in(1, 1)
    do_chunk(0, 0, True, wait_prev_out=False)
    do_chunk(1, 1, True)
    do_chunk(2, 2, True)

    def mid(i, carry):
        n = i * _NBUF
        for k in range(_NBUF):
            do_chunk(n + k, k, True)
        return carry

    lax.fori_loop(1, _NCH // _NBUF, mid, 0)

    do_chunk(_NCH - 2, (_NCH - 2) % _NBUF, False)
    do_chunk(_NCH - 1, (_NCH - 1) % _NBUF, False)

    wait_out((_NCH - 3) % _NBUF)
    wait_out((_NCH - 2) % _NBUF)
    wait_out((_NCH - 1) % _NBUF)


@jax.jit
def _sc_call(x2, ids, tab):
    mesh = plsc.VectorSubcoreMesh(core_axis_name="c", subcore_axis_name="s")
    fn = functools.partial(
        pl.kernel,
        mesh=mesh,
        out_type=jax.ShapeDtypeStruct((_N, _D), jnp.float32),
        scratch_types=[
            pltpu.VMEM((2 * _D,), jnp.float32),
            pltpu.VMEM((_TPW,), jnp.int32),
            pltpu.VMEM((_T, _D), jnp.float32),
            pltpu.VMEM((_T, _D), jnp.float32),
            pltpu.VMEM((_T, _D), jnp.float32),
            pltpu.SemaphoreType.DMA,
            pltpu.SemaphoreType.DMA,
            pltpu.SemaphoreType.DMA,
            pltpu.SemaphoreType.DMA,
            pltpu.SemaphoreType.DMA,
            pltpu.SemaphoreType.DMA,
        ],
    )(_sc_body)
    return fn(x2, ids, tab)


def kernel(x, segment_input_ids, seg_table):
    B, S, D = x.shape
    x2 = x.reshape(B * S, D)
    ids = segment_input_ids.astype(jnp.int32).reshape(-1)
    tab = seg_table.reshape(-1)
    out = _sc_call(x2, ids, tab)
    return out.reshape(B, S, D)


# speedup vs baseline: 1.0435x; 1.0109x over previous
import functools

import jax
import jax.numpy as jnp
from jax import lax
from jax.experimental import pallas as pl
from jax.experimental.pallas import tpu as pltpu
from jax.experimental.pallas import tpu_sc as plsc

_D = 1024
_N = 4 * 8192
_NC = 2
_NS = 16
_NW = _NC * _NS
_TPW = _N // _NW
_T = 32
_NCH = _TPW // _T
_NBUF = 3
_PROBE_NO_COMPUTE = False


def _sc_body(x_hbm, ids_hbm, tab_hbm, out_hbm,
             tab_v, ids_v, b0, b1, b2,
             si0, si1, si2, so0, so1, so2):
    bufs = (b0, b1, b2)
    in_sems = (si0, si1, si2)
    out_sems = (so0, so1, so2)
    wid = lax.axis_index("s") * _NC + lax.axis_index("c")
    base = wid * _TPW

    def start_in(c, k):
        pltpu.async_copy(x_hbm.at[pl.ds(base + c * _T, _T)], bufs[k],
                         in_sems[k])

    def wait_in(k):
        pltpu.make_async_copy(x_hbm.at[pl.ds(0, _T)], bufs[k],
                              in_sems[k]).wait()

    def start_out_half(c, k, h):
        pltpu.async_copy(bufs[k].at[pl.ds(h * 16, 16)],
                         out_hbm.at[pl.ds(base + c * _T + h * 16, 16)],
                         out_sems[k])

    def wait_out(k):
        for _ in range(_T // 16):
            pltpu.make_async_copy(bufs[k].at[pl.ds(0, 16)],
                                  out_hbm.at[pl.ds(0, 16)],
                                  out_sems[k]).wait()

    def compute(c, k):
        buf = bufs[k]
        dnums = lax.GatherDimensionNumbers(
            offset_dims=(), collapsed_slice_dims=(0,), start_index_map=(0,))
        for h in range(_T // 16):
            if _PROBE_NO_COMPUTE:
                start_out_half(c, k, h)
                continue
            m16 = ids_v[pl.ds(c * _T + h * 16, 16)].astype(jnp.float32)
            msps = [lax.gather(m16, jnp.full((16, 1), l, jnp.int32), dnums,
                               (1,),
                               mode=lax.GatherScatterMode.PROMISE_IN_BOUNDS)
                    for l in range(16)]

            def jbody(j, carry):
                sl = pl.ds(j * 16, 16)
                r0 = tab_v[sl]
                r1 = tab_v[pl.ds(_D + j * 16, 16)]
                d = r1 - r0
                for l in range(16):
                    plsc.addupdate(buf.at[h * 16 + l, sl], r0 + msps[l] * d)
                return carry

            lax.fori_loop(0, _D // 16, jbody, 0)
            start_out_half(c, k, h)

    def do_chunk(c, k, start_next_in, wait_prev_out=True):
        wait_in(k)
        compute(c, k)
        if start_next_in:
            k2 = (k + 2) % _NBUF
            if wait_prev_out:
                wait_out(k2)
            start_in(c + 2, k2)

    start_in(0, 0)
    start_in(1, 1)
    pltpu.sync_copy(tab_hbm, tab_v)
    pltpu.sync_copy(ids_hbm.at[pl.ds(base, _TPW)], ids_v)

    do_chunk(0, 0, True, wait_prev_out=False)
    do_chunk(1, 1, True)
    do_chunk(2, 2, True)

    def mid(i, carry):
        n = i * _NBUF
        for k in range(_NBUF):
            do_chunk(n + k, k, True)
        return carry

    lax.fori_loop(1, _NCH // _NBUF, mid, 0)

    do_chunk(_NCH - 2, (_NCH - 2) % _NBUF, False)
    do_chunk(_NCH - 1, (_NCH - 1) % _NBUF, False)

    wait_out((_NCH - 3) % _NBUF)
    wait_out((_NCH - 2) % _NBUF)
    wait_out((_NCH - 1) % _NBUF)


@jax.jit
def _sc_call(x2, ids, tab):
    mesh = plsc.VectorSubcoreMesh(core_axis_name="c", subcore_axis_name="s")
    fn = functools.partial(
        pl.kernel,
        mesh=mesh,
        out_type=jax.ShapeDtypeStruct((_N, _D), jnp.float32),
        scratch_types=[
            pltpu.VMEM((2 * _D,), jnp.float32),
            pltpu.VMEM((_TPW,), jnp.int32),
            pltpu.VMEM((_T, _D), jnp.float32),
            pltpu.VMEM((_T, _D), jnp.float32),
            pltpu.VMEM((_T, _D), jnp.float32),
            pltpu.SemaphoreType.DMA,
            pltpu.SemaphoreType.DMA,
            pltpu.SemaphoreType.DMA,
            pltpu.SemaphoreType.DMA,
            pltpu.SemaphoreType.DMA,
            pltpu.SemaphoreType.DMA,
        ],
    )(_sc_body)
    return fn(x2, ids, tab)


def kernel(x, segment_input_ids, seg_table):
    B, S, D = x.shape
    x2 = x.reshape(B * S, D)
    ids = segment_input_ids.astype(jnp.int32).reshape(-1)
    tab = seg_table.reshape(-1)
    out = _sc_call(x2, ids, tab)
    return out.reshape(B, S, D)


# speedup vs baseline: 1.0492x; 1.0055x over previous
import functools

import jax
import jax.numpy as jnp
from jax import lax
from jax.experimental import pallas as pl
from jax.experimental.pallas import tpu as pltpu
from jax.experimental.pallas import tpu_sc as plsc

_D = 1024
_N = 4 * 8192
_NC = 2
_NS = 16
_NW = _NC * _NS
_TPW = _N // _NW
_T = 16
_NCH = _TPW // _T
_NBUF = 6
_PF = 4


def _sc_body(x_hbm, ids_hbm, tab_hbm, out_hbm,
             tab_v, ids_v, b0, b1, b2, b3, b4, b5,
             si0, si1, si2, si3, si4, si5,
             so0, so1, so2, so3, so4, so5):
    bufs = (b0, b1, b2, b3, b4, b5)
    in_sems = (si0, si1, si2, si3, si4, si5)
    out_sems = (so0, so1, so2, so3, so4, so5)
    wid = lax.axis_index("s") * _NC + lax.axis_index("c")
    base = wid * _TPW

    def start_in(c, k):
        pltpu.async_copy(x_hbm.at[pl.ds(base + c * _T, _T)], bufs[k],
                         in_sems[k])

    def wait_in(k):
        pltpu.make_async_copy(x_hbm.at[pl.ds(0, _T)], bufs[k],
                              in_sems[k]).wait()

    def start_out(c, k):
        pltpu.async_copy(bufs[k], out_hbm.at[pl.ds(base + c * _T, _T)],
                         out_sems[k])

    def wait_out(k):
        pltpu.make_async_copy(bufs[k], out_hbm.at[pl.ds(0, _T)],
                              out_sems[k]).wait()

    def compute(c, k):
        buf = bufs[k]
        dnums = lax.GatherDimensionNumbers(
            offset_dims=(), collapsed_slice_dims=(0,), start_index_map=(0,))
        m16 = ids_v[pl.ds(c * _T, _T)].astype(jnp.float32)
        msps = [lax.gather(m16, jnp.full((16, 1), l, jnp.int32), dnums, (1,),
                           mode=lax.GatherScatterMode.PROMISE_IN_BOUNDS)
                for l in range(16)]

        def jbody(j, carry):
            sl = pl.ds(j * 16, 16)
            r0 = tab_v[sl]
            r1 = tab_v[pl.ds(_D + j * 16, 16)]
            d = r1 - r0
            for l in range(16):
                plsc.addupdate(buf.at[l, sl], r0 + msps[l] * d)
            return carry

        lax.fori_loop(0, _D // 16, jbody, 0)

    def do_chunk(c, k, start_next_in, wait_prev_out=True):
        wait_in(k)
        compute(c, k)
        start_out(c, k)
        if start_next_in:
            k2 = (k + _PF) % _NBUF
            if wait_prev_out:
                wait_out(k2)
            start_in(c + _PF, k2)

    start_in(0, 0)
    start_in(1, 1)
    start_in(2, 2)
    start_in(3, 3)
    pltpu.sync_copy(tab_hbm, tab_v)
    pltpu.sync_copy(ids_hbm.at[pl.ds(base, _TPW)], ids_v)

    do_chunk(0, 0, True, wait_prev_out=False)
    do_chunk(1, 1, True, wait_prev_out=False)
    do_chunk(2, 2, True)
    do_chunk(3, 3, True)

    def mid(i, carry):
        n = 6 * i - 2
        for r in range(_NBUF):
            do_chunk(n + r, (4 + r) % _NBUF, True)
        return carry

    lax.fori_loop(1, 10, mid, 0)

    do_chunk(_NCH - 6, (_NCH - 6) % _NBUF, True)
    do_chunk(_NCH - 5, (_NCH - 5) % _NBUF, True)
    do_chunk(_NCH - 4, (_NCH - 4) % _NBUF, False)
    do_chunk(_NCH - 3, (_NCH - 3) % _NBUF, False)
    do_chunk(_NCH - 2, (_NCH - 2) % _NBUF, False)
    do_chunk(_NCH - 1, (_NCH - 1) % _NBUF, False)

    for c in range(_NCH - 6, _NCH):
        wait_out(c % _NBUF)


@jax.jit
def _sc_call(x2, ids, tab):
    mesh = plsc.VectorSubcoreMesh(core_axis_name="c", subcore_axis_name="s")
    fn = functools.partial(
        pl.kernel,
        mesh=mesh,
        out_type=jax.ShapeDtypeStruct((_N, _D), jnp.float32),
        scratch_types=(
            [pltpu.VMEM((2 * _D,), jnp.float32),
             pltpu.VMEM((_TPW,), jnp.int32)]
            + [pltpu.VMEM((_T, _D), jnp.float32)] * _NBUF
            + [pltpu.SemaphoreType.DMA] * (2 * _NBUF)
        ),
    )(_sc_body)
    return fn(x2, ids, tab)


def kernel(x, segment_input_ids, seg_table):
    B, S, D = x.shape
    x2 = x.reshape(B * S, D)
    ids = segment_input_ids.astype(jnp.int32).reshape(-1)
    tab = seg_table.reshape(-1)
    out = _sc_call(x2, ids, tab)
    return out.reshape(B, S, D)
